# Initial kernel scaffold; baseline (speedup 1.0000x reference)
import jax
import jax.numpy as jnp
from jax.experimental import pallas as pl


def kernel(encoded_tokens, pos_table):
    raise NotImplementedError("write your pallas kernel here")


# speedup vs baseline: 1.7981x; 1.7981x over previous
import jax
import jax.numpy as jnp
from jax.experimental import pallas as pl

B, N_TOKENS, EMBED_DIM = 4, 8192, 768
BLOCK_N = 512


def _add_body(tok_ref, pos_ref, out_ref):
    out_ref[...] = tok_ref[...] + pos_ref[...][jnp.newaxis, :, :]


def kernel(encoded_tokens, pos_table):
    grid = (N_TOKENS // BLOCK_N,)
    return pl.pallas_call(
        _add_body,
        grid=grid,
        in_specs=[
            pl.BlockSpec((B, BLOCK_N, EMBED_DIM), lambda i: (0, i, 0)),
            pl.BlockSpec((BLOCK_N, EMBED_DIM), lambda i: (i, 0)),
        ],
        out_specs=pl.BlockSpec((B, BLOCK_N, EMBED_DIM), lambda i: (0, i, 0)),
        out_shape=jax.ShapeDtypeStruct((B, N_TOKENS, EMBED_DIM), jnp.float32),
    )(encoded_tokens, pos_table)


# speedup vs baseline: 1.7988x; 1.0004x over previous
import jax
import jax.numpy as jnp
from jax.experimental import pallas as pl

B, N_TOKENS, EMBED_DIM = 4, 8192, 768
BLOCK_N = 1024


def _add_body(tok_ref, pos_ref, out_ref):
    out_ref[...] = tok_ref[...] + pos_ref[...][jnp.newaxis, :, :]


def kernel(encoded_tokens, pos_table):
    grid = (N_TOKENS // BLOCK_N,)
    return pl.pallas_call(
        _add_body,
        grid=grid,
        in_specs=[
            pl.BlockSpec((B, BLOCK_N, EMBED_DIM), lambda i: (0, i, 0)),
            pl.BlockSpec((BLOCK_N, EMBED_DIM), lambda i: (i, 0)),
        ],
        out_specs=pl.BlockSpec((B, BLOCK_N, EMBED_DIM), lambda i: (0, i, 0)),
        out_shape=jax.ShapeDtypeStruct((B, N_TOKENS, EMBED_DIM), jnp.float32),
    )(encoded_tokens, pos_table)
